# baseline (device time: 28777 ns/iter reference)
import jax
import jax.numpy as jnp
from jax import lax
from jax.experimental import pallas as pl
from jax.experimental.pallas import tpu as pltpu

N_DEV = 8
M_PER = 128
K = 1024
N_PER = 128

N_SUB = 4
SUB_ROWS = M_PER // N_SUB
HOPS = 4

_GELU_C = 0.7978845608028654


def _gelu(y):
    return 0.5 * y * (1.0 + jnp.tanh(_GELU_C * (y + 0.044715 * y * y * y)))


def _ring(s):
    s = s % N_DEV
    return jnp.where(s < 4, s, 11 - s)


_LAST_HOP_SUBS = {"cw": (0, 1), "ccw": (2, 3)}


def _subs_for_hop(direction, hop):
    return _LAST_HOP_SUBS[direction] if hop == HOPS - 1 else (0, 1, 2, 3)


def kernel(x, w_mat):
    def body(x_ref, w_ref, out_ref, comm_ref,
             cw_send, cw_recv, ccw_send, ccw_recv):
        my_pos = lax.axis_index("i")
        my_slot = _ring(my_pos)
        right = _ring(my_slot + 1)
        left = _ring(my_slot - 1)

        sems = {"cw": (cw_send, cw_recv), "ccw": (ccw_send, ccw_recv)}
        target = {"cw": right, "ccw": left}
        source = {"cw": left, "ccw": right}

        for j in range(N_SUB):
            comm_ref[my_pos, j, :, :] = x_ref[j * SUB_ROWS:(j + 1) * SUB_ROWS, :]

        barrier_sem = pltpu.get_barrier_semaphore()
        for nbr in (left, right):
            pl.semaphore_signal(
                barrier_sem, inc=1,
                device_id=(nbr,), device_id_type=pl.DeviceIdType.MESH,
            )
        pl.semaphore_wait(barrier_sem, 2)

        pending_sends = []

        def start_send(direction, hop, origin, sub):
            send_sem, recv_sem = sems[direction]
            desc = pltpu.make_async_remote_copy(
                src_ref=comm_ref.at[origin, sub],
                dst_ref=comm_ref.at[origin, sub],
                send_sem=send_sem.at[hop, sub],
                recv_sem=recv_sem.at[hop, sub],
                device_id=(target[direction],),
                device_id_type=pl.DeviceIdType.MESH,
            )
            desc.start()
            pending_sends.append(desc)

        for j in range(N_SUB):
            start_send("cw", 0, my_pos, j)
        for j in range(N_SUB):
            start_send("ccw", 0, my_pos, j)

        y = jnp.dot(x_ref[...], w_ref[...], preferred_element_type=jnp.float32)
        out_ref[pl.ds(my_pos * M_PER, M_PER), :] = _gelu(y)

        for h in range(HOPS):
            origins = {
                "cw": _ring(my_slot - h - 1),
                "ccw": _ring(my_slot + h + 1),
            }
            if h == HOPS - 1:
                for d in ("cw", "ccw"):
                    for dist in (1, 2, 3):
                        o = _ring(my_slot - dist) if d == "cw" else (
                            _ring(my_slot + dist))
                        y = jnp.dot(comm_ref[o].reshape(M_PER, K), w_ref[...],
                                    preferred_element_type=jnp.float32)
                        out_ref[pl.ds(o * M_PER, M_PER), :] = _gelu(y)
            for j in range(N_SUB):
                for d in ("cw", "ccw"):
                    subs = _subs_for_hop(d, h)
                    if j >= len(subs):
                        continue
                    sub = subs[j]
                    send_sem, recv_sem = sems[d]
                    recv = pltpu.make_async_remote_copy(
                        src_ref=comm_ref.at[origins[d], sub],
                        dst_ref=comm_ref.at[origins[d], sub],
                        send_sem=send_sem.at[h, sub],
                        recv_sem=recv_sem.at[h, sub],
                        device_id=(source[d],),
                        device_id_type=pl.DeviceIdType.MESH,
                    )
                    recv.wait_recv()
                    if h + 1 < HOPS and sub in _subs_for_hop(d, h + 1):
                        start_send(d, h + 1, origins[d], sub)

        o4 = _ring(my_slot - 4)
        y = jnp.dot(comm_ref[o4].reshape(M_PER, K), w_ref[...],
                    preferred_element_type=jnp.float32)
        out_ref[pl.ds(o4 * M_PER, M_PER), :] = _gelu(y)

        for desc in pending_sends:
            desc.wait_send()

    return pl.pallas_call(
        body,
        out_shape=jax.ShapeDtypeStruct((N_DEV * M_PER, N_PER), jnp.float32),
        in_specs=[
            pl.BlockSpec(memory_space=pltpu.VMEM),
            pl.BlockSpec(memory_space=pltpu.VMEM),
        ],
        out_specs=pl.BlockSpec(memory_space=pltpu.VMEM),
        scratch_shapes=[
            pltpu.VMEM((N_DEV, N_SUB, SUB_ROWS, K), jnp.float32),
            pltpu.SemaphoreType.DMA((HOPS, N_SUB)),
            pltpu.SemaphoreType.DMA((HOPS, N_SUB)),
            pltpu.SemaphoreType.DMA((HOPS, N_SUB)),
            pltpu.SemaphoreType.DMA((HOPS, N_SUB)),
        ],
        compiler_params=pltpu.CompilerParams(collective_id=0),
    )(x, w_mat)


# device time: 28735 ns/iter; 1.0015x vs baseline; 1.0015x over previous
import jax
import jax.numpy as jnp
from jax import lax
from jax.experimental import pallas as pl
from jax.experimental.pallas import tpu as pltpu

N_DEV = 8
M_PER = 128
K = 1024
N_PER = 128

N_SUB = 4
SUB_ROWS = M_PER // N_SUB
HOPS = 4

_GELU_C = 0.7978845608028654


def _gelu(y):
    return 0.5 * y * (1.0 + jnp.tanh(_GELU_C * (y + 0.044715 * y * y * y)))


def _ring(s):
    s = s % N_DEV
    return jnp.where(s < 4, s, 11 - s)


_LAST_HOP_SUBS = {"cw": (0, 1), "ccw": (2, 3)}


def _subs_for_hop(direction, hop):
    return _LAST_HOP_SUBS[direction] if hop == HOPS - 1 else (0, 1, 2, 3)


def kernel(x, w_mat):
    def body(x_ref, w_ref, out_ref, comm_ref,
             cw_send, cw_recv, ccw_send, ccw_recv):
        my_pos = lax.axis_index("i")
        my_slot = _ring(my_pos)
        right = _ring(my_slot + 1)
        left = _ring(my_slot - 1)

        sems = {"cw": (cw_send, cw_recv), "ccw": (ccw_send, ccw_recv)}
        target = {"cw": right, "ccw": left}
        source = {"cw": left, "ccw": right}

        barrier_sem = pltpu.get_barrier_semaphore()
        for nbr in (left, right):
            pl.semaphore_signal(
                barrier_sem, inc=1,
                device_id=(nbr,), device_id_type=pl.DeviceIdType.MESH,
            )
        pl.semaphore_wait(barrier_sem, 2)

        pending_sends = []

        def start_send(direction, hop, origin, sub, src=None):
            send_sem, recv_sem = sems[direction]
            desc = pltpu.make_async_remote_copy(
                src_ref=comm_ref.at[origin, sub] if src is None else src,
                dst_ref=comm_ref.at[origin, sub],
                send_sem=send_sem.at[hop, sub],
                recv_sem=recv_sem.at[hop, sub],
                device_id=(target[direction],),
                device_id_type=pl.DeviceIdType.MESH,
            )
            desc.start()
            pending_sends.append(desc)

        for j in range(N_SUB):
            start_send("cw", 0, my_pos, j,
                       src=x_ref.at[pl.ds(j * SUB_ROWS, SUB_ROWS)])
        for j in range(N_SUB):
            start_send("ccw", 0, my_pos, j,
                       src=x_ref.at[pl.ds(j * SUB_ROWS, SUB_ROWS)])

        y = jnp.dot(x_ref[...], w_ref[...], preferred_element_type=jnp.float32)
        out_ref[pl.ds(my_pos * M_PER, M_PER), :] = _gelu(y)

        for h in range(HOPS):
            origins = {
                "cw": _ring(my_slot - h - 1),
                "ccw": _ring(my_slot + h + 1),
            }
            if h == HOPS - 1:
                for d in ("cw", "ccw"):
                    for dist in (1, 2, 3):
                        o = _ring(my_slot - dist) if d == "cw" else (
                            _ring(my_slot + dist))
                        y = jnp.dot(comm_ref[o].reshape(M_PER, K), w_ref[...],
                                    preferred_element_type=jnp.float32)
                        out_ref[pl.ds(o * M_PER, M_PER), :] = _gelu(y)
            for j in range(N_SUB):
                for d in ("cw", "ccw"):
                    subs = _subs_for_hop(d, h)
                    if j >= len(subs):
                        continue
                    sub = subs[j]
                    send_sem, recv_sem = sems[d]
                    recv = pltpu.make_async_remote_copy(
                        src_ref=comm_ref.at[origins[d], sub],
                        dst_ref=comm_ref.at[origins[d], sub],
                        send_sem=send_sem.at[h, sub],
                        recv_sem=recv_sem.at[h, sub],
                        device_id=(source[d],),
                        device_id_type=pl.DeviceIdType.MESH,
                    )
                    recv.wait_recv()
                    if h + 1 < HOPS and sub in _subs_for_hop(d, h + 1):
                        start_send(d, h + 1, origins[d], sub)

        o4 = _ring(my_slot - 4)
        y = jnp.dot(comm_ref[o4].reshape(M_PER, K), w_ref[...],
                    preferred_element_type=jnp.float32)
        out_ref[pl.ds(o4 * M_PER, M_PER), :] = _gelu(y)

        for desc in pending_sends:
            desc.wait_send()

    return pl.pallas_call(
        body,
        out_shape=jax.ShapeDtypeStruct((N_DEV * M_PER, N_PER), jnp.float32),
        in_specs=[
            pl.BlockSpec(memory_space=pltpu.VMEM),
            pl.BlockSpec(memory_space=pltpu.VMEM),
        ],
        out_specs=pl.BlockSpec(memory_space=pltpu.VMEM),
        scratch_shapes=[
            pltpu.VMEM((N_DEV, N_SUB, SUB_ROWS, K), jnp.float32),
            pltpu.SemaphoreType.DMA((HOPS, N_SUB)),
            pltpu.SemaphoreType.DMA((HOPS, N_SUB)),
            pltpu.SemaphoreType.DMA((HOPS, N_SUB)),
            pltpu.SemaphoreType.DMA((HOPS, N_SUB)),
        ],
        compiler_params=pltpu.CompilerParams(collective_id=0),
    )(x, w_mat)


# device time: 25861 ns/iter; 1.1128x vs baseline; 1.1111x over previous
import jax
import jax.numpy as jnp
from jax import lax
from jax.experimental import pallas as pl
from jax.experimental.pallas import tpu as pltpu

N_DEV = 8
M_PER = 128
K = 1024
N_PER = 128

N_SUB = 4
SUB_ROWS = M_PER // N_SUB

_GELU_C = 0.7978845608028654


def _gelu(y):
    return 0.5 * y * (1.0 + jnp.tanh(_GELU_C * (y + 0.044715 * y * y * y)))


def _ring(s):
    s = s % N_DEV
    return jnp.where(s < 4, s, 11 - s)


def kernel(x, w_mat):
    def body(x_ref, w_ref, out_ref, comm_ref,
             h_send, h_recv, l_send, l_recv, c_send, c_recv):
        my_pos = lax.axis_index("i")
        my_slot = _ring(my_pos)
        sign = 1 - 2 * (my_slot % 2)

        partner = {
            "h": _ring(my_slot - sign),
            "l": _ring(my_slot + sign),
            "c": _ring(my_slot + 3 * sign),
        }
        sems = {"h": (h_send, h_recv), "l": (l_send, l_recv),
                "c": (c_send, c_recv)}
        o_h1, o_h2, o_h3 = (partner["h"], _ring(my_slot - 2 * sign),
                            _ring(my_slot - 3 * sign))
        o_l1, o_l2 = partner["l"], _ring(my_slot + 2 * sign)
        o_c1, o_c2 = partner["c"], _ring(my_slot + 4 * sign)

        barrier_sem = pltpu.get_barrier_semaphore()
        for nbr in partner.values():
            pl.semaphore_signal(
                barrier_sem, inc=1,
                device_id=(nbr,), device_id_type=pl.DeviceIdType.MESH,
            )
        pl.semaphore_wait(barrier_sem, 3)

        pending_sends = []

        def start_send(link, depth, origin, sub, src=None):
            send_sem, recv_sem = sems[link]
            desc = pltpu.make_async_remote_copy(
                src_ref=comm_ref.at[origin, sub] if src is None else src,
                dst_ref=comm_ref.at[origin, sub],
                send_sem=send_sem.at[depth, sub],
                recv_sem=recv_sem.at[depth, sub],
                device_id=(partner[link],),
                device_id_type=pl.DeviceIdType.MESH,
            )
            desc.start()
            pending_sends.append(desc)

        def wait_recv(link, depth, origin, sub):
            send_sem, recv_sem = sems[link]
            desc = pltpu.make_async_remote_copy(
                src_ref=comm_ref.at[origin, sub],
                dst_ref=comm_ref.at[origin, sub],
                send_sem=send_sem.at[depth, sub],
                recv_sem=recv_sem.at[depth, sub],
                device_id=(partner[link],),
                device_id_type=pl.DeviceIdType.MESH,
            )
            desc.wait_recv()

        def gemm(origin):
            y = jnp.dot(comm_ref[origin].reshape(M_PER, K), w_ref[...],
                        preferred_element_type=jnp.float32)
            out_ref[pl.ds(origin * M_PER, M_PER), :] = _gelu(y)

        for j in range(N_SUB):
            xsub = x_ref.at[pl.ds(j * SUB_ROWS, SUB_ROWS)]
            start_send("h", 0, my_pos, j, src=xsub)
            start_send("l", 0, my_pos, j, src=xsub)
            start_send("c", 0, my_pos, j, src=xsub)

        y = jnp.dot(x_ref[...], w_ref[...], preferred_element_type=jnp.float32)
        out_ref[pl.ds(my_pos * M_PER, M_PER), :] = _gelu(y)

        for j in range(N_SUB):
            wait_recv("l", 0, o_l1, j)
            start_send("h", 1, o_l1, j)
            wait_recv("h", 0, o_h1, j)
            start_send("l", 1, o_h1, j)
            start_send("c", 1, o_h1, j)

        for j in range(N_SUB):
            wait_recv("l", 1, o_l2, j)
            start_send("h", 2, o_l2, j)

        gemm(o_l1)
        gemm(o_h1)
        for j in range(N_SUB):
            wait_recv("c", 0, o_c1, j)
        gemm(o_c1)
        gemm(o_l2)
        for j in range(N_SUB):
            wait_recv("h", 1, o_h2, j)
        gemm(o_h2)
        for j in range(N_SUB):
            wait_recv("c", 1, o_c2, j)
        gemm(o_c2)
        for j in range(N_SUB):
            wait_recv("h", 2, o_h3, j)
        gemm(o_h3)

        for desc in pending_sends:
            desc.wait_send()

    return pl.pallas_call(
        body,
        out_shape=jax.ShapeDtypeStruct((N_DEV * M_PER, N_PER), jnp.float32),
        in_specs=[
            pl.BlockSpec(memory_space=pltpu.VMEM),
            pl.BlockSpec(memory_space=pltpu.VMEM),
        ],
        out_specs=pl.BlockSpec(memory_space=pltpu.VMEM),
        scratch_shapes=[
            pltpu.VMEM((N_DEV, N_SUB, SUB_ROWS, K), jnp.float32),
            pltpu.SemaphoreType.DMA((3, N_SUB)),
            pltpu.SemaphoreType.DMA((3, N_SUB)),
            pltpu.SemaphoreType.DMA((2, N_SUB)),
            pltpu.SemaphoreType.DMA((2, N_SUB)),
            pltpu.SemaphoreType.DMA((2, N_SUB)),
            pltpu.SemaphoreType.DMA((2, N_SUB)),
        ],
        compiler_params=pltpu.CompilerParams(collective_id=0),
    )(x, w_mat)
